# baseline (speedup 1.0000x reference)
import functools

import jax
import jax.numpy as jnp
from jax import lax
from jax.experimental import pallas as pl
from jax.experimental.pallas import tpu as pltpu
from jax.experimental.pallas import tpu_sc as plsc

N = 10000
D = 128
H = 64
NG = 64
NC = 2
NS = 16
NW = NC * NS
CW = 128
GW = 1024
SUB = GW // CW
NPAD = 10112
RPS = NPAD // NS
DEGW = 16
Q0 = 96
Q1 = 61


def _deg_body(dst_hbm, ones_hbm, out_hbm, idx_v, ones_v, zb_v, acc_sh):
    cid = lax.axis_index("c")
    sid = lax.axis_index("s")
    wid = sid * NC + cid
    ch = jnp.where(cid == 0, Q0, Q1)
    pltpu.sync_copy(dst_hbm.at[wid], idx_v)
    pltpu.sync_copy(ones_hbm, ones_v)

    def zero(i, c):
        zb_v[i] = jnp.zeros((DEGW,), jnp.float32)
        return c

    lax.fori_loop(0, RPS, zero, 0)
    pltpu.sync_copy(zb_v, acc_sh.at[pl.ds(sid * RPS, RPS)])
    plsc.subcore_barrier()

    def chunk(c, carry):
        pltpu.sync_copy(ones_v, acc_sh.at[idx_v.at[c]], add=True)
        return carry

    lax.fori_loop(0, ch, chunk, 0)
    plsc.subcore_barrier()
    pltpu.sync_copy(acc_sh.at[pl.ds(sid * RPS, RPS)], zb_v)
    pltpu.sync_copy(zb_v, out_hbm.at[cid, pl.ds(sid * RPS, RPS)])


def _spmm_body(z_hbm, src_hbm, dst_hbm, out_hbm, sidx_v, didx_v, rows_v, slab_v,
               acc_sh):
    cid = lax.axis_index("c")
    sid = lax.axis_index("s")
    wid = sid * NC + cid
    nch = jnp.where(cid == 0, Q0, Q1)
    pltpu.sync_copy(src_hbm.at[wid], sidx_v)
    pltpu.sync_copy(dst_hbm.at[wid], didx_v)

    def zero(i, c):
        for j in range(H // 16):
            slab_v[i, pl.ds(16 * j, 16)] = jnp.zeros((16,), jnp.float32)
        return c

    lax.fori_loop(0, RPS, zero, 0)
    pltpu.sync_copy(slab_v, acc_sh.at[pl.ds(sid * RPS, RPS)])
    plsc.subcore_barrier()

    def chunk(c, carry):
        pltpu.sync_copy(z_hbm.at[sidx_v.at[c]], rows_v)
        pltpu.sync_copy(rows_v, acc_sh.at[didx_v.at[c]], add=True)
        return carry

    lax.fori_loop(0, nch, chunk, 0)
    plsc.subcore_barrier()
    pltpu.sync_copy(acc_sh.at[pl.ds(sid * RPS, RPS)], slab_v)
    pltpu.sync_copy(slab_v, out_hbm.at[cid, pl.ds(sid * RPS, RPS)])


@functools.cache
def _sc_mesh():
    return plsc.VectorSubcoreMesh(
        core_axis_name="c", subcore_axis_name="s", num_cores=NC, num_subcores=NS)


def _make_deg(ch):
    return pl.kernel(
        _deg_body,
        out_type=jax.ShapeDtypeStruct((NC, NPAD, DEGW), jnp.float32),
        mesh=_sc_mesh(),
        compiler_params=pltpu.CompilerParams(use_tc_tiling_on_sc=False),
        scratch_types=[
            pltpu.VMEM((ch, CW), jnp.int32),
            pltpu.VMEM((CW, DEGW), jnp.float32),
            pltpu.VMEM((RPS, DEGW), jnp.float32),
            pltpu.VMEM_SHARED((NPAD, DEGW), jnp.float32),
        ],
    )


def _make_spmm(nch):
    return pl.kernel(
        _spmm_body,
        out_type=jax.ShapeDtypeStruct((NC, NPAD, H), jnp.float32),
        mesh=_sc_mesh(),
        compiler_params=pltpu.CompilerParams(use_tc_tiling_on_sc=False),
        scratch_types=[
            pltpu.VMEM((nch, CW), jnp.int32),
            pltpu.VMEM((nch, CW), jnp.int32),
            pltpu.VMEM((CW, H), jnp.float32),
            pltpu.VMEM((RPS, H), jnp.float32),
            pltpu.VMEM_SHARED((NPAD, H), jnp.float32),
        ],
    )


def _tc1_body(dp_ref, x_ref, w_ref, z_ref, dinv_ref):
    deg = dp_ref[0, :N, 0:1] + dp_ref[1, :N, 0:1] + 1.0
    dinv = lax.rsqrt(deg)
    y = jnp.dot(x_ref[...], w_ref[...], preferred_element_type=jnp.float32)
    z_ref[...] = y * dinv
    dinv_ref[...] = dinv


def _tc2_body(p_ref, z_ref, dinv_ref, b_ref, w_ref, zo_ref):
    di = dinv_ref[...]
    s = p_ref[0, :N] + p_ref[1, :N] + z_ref[...]
    h = jnp.maximum(di * s + b_ref[...], 0.0)
    zo_ref[...] = jnp.dot(h, w_ref[...], preferred_element_type=jnp.float32) * di


def _tc3_body(p_ref, z_ref, dinv_ref, b_ref, batch_ref, a1_ref, ab1_ref,
              a2_ref, ab2_ref, out_ref, h_scr, g_scr):
    di = dinv_ref[...]
    s = p_ref[0, :N] + p_ref[1, :N] + z_ref[...]
    h_scr[...] = jnp.maximum(di * s + b_ref[...], 0.0)

    def seg(g, carry):
        m = batch_ref[...] == g
        mx = jnp.max(jnp.where(m, h_scr[...], -jnp.inf), axis=0, keepdims=True)
        g_scr[pl.ds(g, 1), :] = mx
        return carry

    lax.fori_loop(0, NG, seg, 0)
    g2 = jnp.maximum(
        jnp.dot(g_scr[...], a1_ref[...], preferred_element_type=jnp.float32)
        + ab1_ref[...], 0.0)
    out_ref[...] = (
        jnp.dot(g2, a2_ref[...], preferred_element_type=jnp.float32)
        + ab2_ref[...])


def kernel(x, edge_index, batch, W1, b1, W2, b2, W3, b3, A1, ab1, A2, ab2):
    assert x.shape == (N, D)
    e = edge_index.shape[1]
    cap = NS * (Q0 + Q1) * CW
    assert cap >= e
    pad_dst = (N + jnp.arange(cap - e, dtype=edge_index.dtype) % (NPAD - N))
    srcf = jnp.concatenate(
        [edge_index[0], jnp.zeros((cap - e,), edge_index.dtype)])
    dstf = jnp.concatenate([edge_index[1], pad_dst])

    def pack(flat):
        p0 = flat[:NS * Q0 * CW].reshape(NS, Q0, CW)
        p1 = flat[NS * Q0 * CW:].reshape(NS, Q1, CW)
        p1 = jnp.pad(p1, ((0, 0), (0, Q0 - Q1), (0, 0)))
        return jnp.stack([p0, p1], axis=1).reshape(NW, Q0, CW)

    src = pack(srcf)
    dst = pack(dstf)
    nch = Q0

    degp = _make_deg(nch)(dst, jnp.ones((CW, DEGW), jnp.float32))

    z1, dinv = pl.pallas_call(
        _tc1_body,
        out_shape=(jax.ShapeDtypeStruct((N, H), jnp.float32),
                   jax.ShapeDtypeStruct((N, 1), jnp.float32)),
    )(degp, x, W1)

    spmm = _make_spmm(nch)
    tc2 = pl.pallas_call(
        _tc2_body, out_shape=jax.ShapeDtypeStruct((N, H), jnp.float32))

    p1 = spmm(z1, src, dst)
    z2 = tc2(p1, z1, dinv, b1.reshape(1, H), W2)
    p2 = spmm(z2, src, dst)
    z3 = tc2(p2, z2, dinv, b2.reshape(1, H), W3)
    p3 = spmm(z3, src, dst)

    out = pl.pallas_call(
        _tc3_body, out_shape=jax.ShapeDtypeStruct((NG, 1), jnp.float32),
        scratch_shapes=[pltpu.VMEM((N, H), jnp.float32),
                        pltpu.VMEM((NG, H), jnp.float32)],
    )(p3, z3, dinv, b3.reshape(1, H), batch.reshape(N, 1), A1,
      ab1.reshape(1, 16), A2, ab2.reshape(1, 1))
    return out


# speedup vs baseline: 1.0356x; 1.0356x over previous
import functools

import jax
import jax.numpy as jnp
from jax import lax
from jax.experimental import pallas as pl
from jax.experimental.pallas import tpu as pltpu
from jax.experimental.pallas import tpu_sc as plsc

N = 10000
D = 128
H = 64
NG = 64
NC = 2
NS = 16
NW = NC * NS
CW = 128
GW = 1024
SUB = GW // CW
NPAD = 10112
RPS = NPAD // NS
DEGW = 16
Q0 = 89
Q1 = 68


def _deg_body(dst_hbm, ones_hbm, out_hbm, idx_v, ones_v, zb_v, acc_sh):
    cid = lax.axis_index("c")
    sid = lax.axis_index("s")
    wid = sid * NC + cid
    ch = jnp.where(cid == 0, Q0, Q1)
    pltpu.sync_copy(dst_hbm.at[wid], idx_v)
    pltpu.sync_copy(ones_hbm, ones_v)

    def zero(i, c):
        zb_v[i] = jnp.zeros((DEGW,), jnp.float32)
        return c

    lax.fori_loop(0, RPS, zero, 0)
    pltpu.sync_copy(zb_v, acc_sh.at[pl.ds(sid * RPS, RPS)])
    plsc.subcore_barrier()

    def chunk(c, carry):
        pltpu.sync_copy(ones_v, acc_sh.at[idx_v.at[c]], add=True)
        return carry

    lax.fori_loop(0, ch, chunk, 0)
    plsc.subcore_barrier()
    pltpu.sync_copy(acc_sh.at[pl.ds(sid * RPS, RPS)], zb_v)
    pltpu.sync_copy(zb_v, out_hbm.at[cid, pl.ds(sid * RPS, RPS)])


def _spmm_body(z_hbm, src_hbm, dst_hbm, out_hbm, sidx_v, didx_v, rows_v, slab_v,
               acc_sh):
    cid = lax.axis_index("c")
    sid = lax.axis_index("s")
    wid = sid * NC + cid
    nch = jnp.where(cid == 0, Q0, Q1)
    pltpu.sync_copy(src_hbm.at[wid], sidx_v)
    pltpu.sync_copy(dst_hbm.at[wid], didx_v)

    def zero(i, c):
        for j in range(H // 16):
            slab_v[i, pl.ds(16 * j, 16)] = jnp.zeros((16,), jnp.float32)
        return c

    lax.fori_loop(0, RPS, zero, 0)
    pltpu.sync_copy(slab_v, acc_sh.at[pl.ds(sid * RPS, RPS)])
    plsc.subcore_barrier()

    def chunk(c, carry):
        pltpu.sync_copy(z_hbm.at[sidx_v.at[c]], rows_v)
        pltpu.sync_copy(rows_v, acc_sh.at[didx_v.at[c]], add=True)
        return carry

    lax.fori_loop(0, nch, chunk, 0)
    plsc.subcore_barrier()
    pltpu.sync_copy(acc_sh.at[pl.ds(sid * RPS, RPS)], slab_v)
    pltpu.sync_copy(slab_v, out_hbm.at[cid, pl.ds(sid * RPS, RPS)])


@functools.cache
def _sc_mesh():
    return plsc.VectorSubcoreMesh(
        core_axis_name="c", subcore_axis_name="s", num_cores=NC, num_subcores=NS)


def _make_deg(ch):
    return pl.kernel(
        _deg_body,
        out_type=jax.ShapeDtypeStruct((NC, NPAD, DEGW), jnp.float32),
        mesh=_sc_mesh(),
        compiler_params=pltpu.CompilerParams(use_tc_tiling_on_sc=False),
        scratch_types=[
            pltpu.VMEM((ch, CW), jnp.int32),
            pltpu.VMEM((CW, DEGW), jnp.float32),
            pltpu.VMEM((RPS, DEGW), jnp.float32),
            pltpu.VMEM_SHARED((NPAD, DEGW), jnp.float32),
        ],
    )


def _make_spmm(nch):
    return pl.kernel(
        _spmm_body,
        out_type=jax.ShapeDtypeStruct((NC, NPAD, H), jnp.float32),
        mesh=_sc_mesh(),
        compiler_params=pltpu.CompilerParams(use_tc_tiling_on_sc=False),
        scratch_types=[
            pltpu.VMEM((nch, CW), jnp.int32),
            pltpu.VMEM((nch, CW), jnp.int32),
            pltpu.VMEM((CW, H), jnp.float32),
            pltpu.VMEM((RPS, H), jnp.float32),
            pltpu.VMEM_SHARED((NPAD, H), jnp.float32),
        ],
    )


def _tc1_body(dp_ref, x_ref, w_ref, z_ref, dinv_ref):
    deg = dp_ref[0, :N, 0:1] + dp_ref[1, :N, 0:1] + 1.0
    dinv = lax.rsqrt(deg)
    y = jnp.dot(x_ref[...], w_ref[...], preferred_element_type=jnp.float32)
    z_ref[...] = y * dinv
    dinv_ref[...] = dinv


def _tc2_body(p_ref, z_ref, dinv_ref, b_ref, w_ref, zo_ref):
    di = dinv_ref[...]
    s = p_ref[0, :N] + p_ref[1, :N] + z_ref[...]
    h = jnp.maximum(di * s + b_ref[...], 0.0)
    zo_ref[...] = jnp.dot(h, w_ref[...], preferred_element_type=jnp.float32) * di


def _tc3_body(p_ref, z_ref, dinv_ref, b_ref, batch_ref, a1_ref, ab1_ref,
              a2_ref, ab2_ref, out_ref, h_scr, g_scr):
    di = dinv_ref[...]
    s = p_ref[0, :N] + p_ref[1, :N] + z_ref[...]
    h_scr[...] = jnp.maximum(di * s + b_ref[...], 0.0)

    def seg(g, carry):
        m = batch_ref[...] == g
        mx = jnp.max(jnp.where(m, h_scr[...], -jnp.inf), axis=0, keepdims=True)
        g_scr[pl.ds(g, 1), :] = mx
        return carry

    lax.fori_loop(0, NG, seg, 0)
    g2 = jnp.maximum(
        jnp.dot(g_scr[...], a1_ref[...], preferred_element_type=jnp.float32)
        + ab1_ref[...], 0.0)
    out_ref[...] = (
        jnp.dot(g2, a2_ref[...], preferred_element_type=jnp.float32)
        + ab2_ref[...])


def kernel(x, edge_index, batch, W1, b1, W2, b2, W3, b3, A1, ab1, A2, ab2):
    assert x.shape == (N, D)
    e = edge_index.shape[1]
    cap = NS * (Q0 + Q1) * CW
    assert cap >= e
    pad_dst = (N + jnp.arange(cap - e, dtype=edge_index.dtype) % (NPAD - N))
    srcf = jnp.concatenate(
        [edge_index[0], jnp.zeros((cap - e,), edge_index.dtype)])
    dstf = jnp.concatenate([edge_index[1], pad_dst])

    def pack(flat):
        p0 = flat[:NS * Q0 * CW].reshape(NS, Q0, CW)
        p1 = flat[NS * Q0 * CW:].reshape(NS, Q1, CW)
        p1 = jnp.pad(p1, ((0, 0), (0, Q0 - Q1), (0, 0)))
        return jnp.stack([p0, p1], axis=1).reshape(NW, Q0, CW)

    src = pack(srcf)
    dst = pack(dstf)
    nch = Q0

    degp = _make_deg(nch)(dst, jnp.ones((CW, DEGW), jnp.float32))

    z1, dinv = pl.pallas_call(
        _tc1_body,
        out_shape=(jax.ShapeDtypeStruct((N, H), jnp.float32),
                   jax.ShapeDtypeStruct((N, 1), jnp.float32)),
    )(degp, x, W1)

    spmm = _make_spmm(nch)
    tc2 = pl.pallas_call(
        _tc2_body, out_shape=jax.ShapeDtypeStruct((N, H), jnp.float32))

    p1 = spmm(z1, src, dst)
    z2 = tc2(p1, z1, dinv, b1.reshape(1, H), W2)
    p2 = spmm(z2, src, dst)
    z3 = tc2(p2, z2, dinv, b2.reshape(1, H), W3)
    p3 = spmm(z3, src, dst)

    out = pl.pallas_call(
        _tc3_body, out_shape=jax.ShapeDtypeStruct((NG, 1), jnp.float32),
        scratch_shapes=[pltpu.VMEM((N, H), jnp.float32),
                        pltpu.VMEM((NG, H), jnp.float32)],
    )(p3, z3, dinv, b3.reshape(1, H), batch.reshape(N, 1), A1,
      ab1.reshape(1, 16), A2, ab2.reshape(1, 1))
    return out
